# baseline (speedup 1.0000x reference)
import functools

import jax
import jax.numpy as jnp
from jax import lax
from jax.experimental import pallas as pl
from jax.experimental.pallas import tpu as pltpu
from jax.experimental.pallas import tpu_sc as plsc

N = 10000
E = 320000
D = 128
NC = 2
NS = 16
NW = NC * NS
CHUNK = 128
CPS = 8
SUP = CPS * CHUNK
PT = ((E + NW - 1) // NW + SUP - 1) // SUP * SUP
EPAD = PT * NW
NCHUNK = PT // CHUNK
NSUP = PT // SUP
NPAD = 10240
STRIPE = NPAD // NS

_mesh = plsc.VectorSubcoreMesh(core_axis_name="c", subcore_axis_name="s",
                               num_cores=NC, num_subcores=NS)


def _zero_fill(buf, nrows, ncols):
    z16 = jnp.zeros((16,), jnp.float32)

    def fill(i, _):
        buf[i // (ncols // 16), pl.ds((i % (ncols // 16)) * 16, 16)] = z16
        return 0

    lax.fori_loop(0, nrows * (ncols // 16), fill, 0)


def _seg_core(src_hbm, dst_hbm, feat_hbm, out_sum, sum_sh,
              sidx, didx, rows0, rows1, gsem0, gsem1):
    c = lax.axis_index("c")
    s = lax.axis_index("s")

    _zero_fill(rows0, CHUNK, D)
    row0 = s * STRIPE

    def zcopy(j, _):
        pltpu.sync_copy(rows0, sum_sh.at[pl.ds(row0 + j * CHUNK, CHUNK), :])
        return 0

    lax.fori_loop(0, STRIPE // CHUNK, zcopy, 0)
    plsc.subcore_barrier()

    g = c * NS + s
    cbase = g * NCHUNK
    pltpu.sync_copy(src_hbm.at[pl.ds(cbase, CPS), :], sidx)
    pltpu.sync_copy(dst_hbm.at[pl.ds(cbase, CPS), :], didx)
    del gsem0, gsem1
    bufs = (rows0, rows1)

    def qbody(q, _):
        for k in range(CPS):
            pltpu.sync_copy(feat_hbm.at[sidx.at[k]], bufs[k % 2])
            pltpu.sync_copy(bufs[k % 2], sum_sh.at[didx.at[k]], add=True)

        @pl.when(q + 1 < NSUP)
        def _():
            nxt = cbase + (q + 1) * CPS
            pltpu.sync_copy(src_hbm.at[pl.ds(nxt, CPS), :], sidx)
            pltpu.sync_copy(dst_hbm.at[pl.ds(nxt, CPS), :], didx)

        return 0

    lax.fori_loop(0, NSUP, qbody, 0)
    plsc.subcore_barrier()

    pltpu.sync_copy(sum_sh.at[pl.ds(row0, STRIPE), :],
                    out_sum.at[c, pl.ds(row0, STRIPE), :])


@functools.partial(
    pl.kernel,
    out_type=jax.ShapeDtypeStruct((NW, NPAD), jnp.float32),
    mesh=_mesh,
    scratch_types=dict(
        didx=pltpu.VMEM((SUP,), jnp.int32),
        cntv=pltpu.VMEM((NPAD,), jnp.float32),
    ),
    compiler_params=pltpu.CompilerParams(needs_layout_passes=False),
)
def _degree(dst_hbm, out_cnt, didx, cntv):
    c = lax.axis_index("c")
    s = lax.axis_index("s")
    z16 = jnp.zeros((16,), jnp.float32)

    def czf(i, _):
        cntv[pl.ds(i * 16, 16)] = z16
        return 0

    lax.fori_loop(0, NPAD // 16, czf, 0)
    g = c * NS + s
    ebase = g * PT
    one16 = jnp.ones((16,), jnp.float32)

    def esup(i, _):
        pltpu.sync_copy(dst_hbm.at[pl.ds(ebase + i * SUP, SUP)], didx)

        def cadd(j, _):
            v = didx[pl.ds(j * 16, 16)]
            plsc.addupdate_scatter(cntv, [v], one16)
            return 0

        lax.fori_loop(0, SUP // 16, cadd, 0)
        return 0

    lax.fori_loop(0, NSUP, esup, 0)
    pltpu.sync_copy(cntv, out_cnt.at[g])


@functools.partial(
    pl.kernel,
    out_type=jax.ShapeDtypeStruct((NC, NPAD, D), jnp.float32),
    mesh=_mesh,
    scratch_types=dict(
        sidx=pltpu.VMEM((CPS, CHUNK), jnp.int32),
        didx=pltpu.VMEM((CPS, CHUNK), jnp.int32),
        rows0=pltpu.VMEM((CHUNK, D), jnp.float32),
        rows1=pltpu.VMEM((CHUNK, D), jnp.float32),
        sum_sh=pltpu.VMEM_SHARED((NPAD, D), jnp.float32),
        gsem0=pltpu.SemaphoreType.DMA,
        gsem1=pltpu.SemaphoreType.DMA,
    ),
)
def _seg_sum(src_hbm, dst_hbm, feat_hbm, out_sum,
             sidx, didx, rows0, rows1, sum_sh, gsem0, gsem1):
    _seg_core(src_hbm, dst_hbm, feat_hbm, out_sum, sum_sh,
              sidx, didx, rows0, rows1, gsem0, gsem1)


BN = 1024


def _layer1_body(p0, p1, cn, x, wl, wr, b, out):
    cnt = jnp.maximum(jnp.sum(cn[...], axis=0), 1.0)
    mean = (p0[...] + p1[...]) / cnt[:, None]
    acc = jnp.dot(mean, wl[...], preferred_element_type=jnp.float32)
    acc = acc + jnp.dot(x[...], wr[...], preferred_element_type=jnp.float32)
    out[...] = jnp.maximum(acc + b[...], 0.0)


def _layer2_body(q0, q1, cn, h, wl, wr, b, wh, bh, out):
    cnt = jnp.maximum(jnp.sum(cn[...], axis=0), 1.0)
    mean = (q0[...] + q1[...]) / cnt[:, None]
    acc = jnp.dot(mean, wl[...], preferred_element_type=jnp.float32)
    acc = acc + jnp.dot(h[...], wr[...], preferred_element_type=jnp.float32)
    h2 = jnp.maximum(acc + b[...], 0.0)
    out[...] = jnp.dot(h2, wh[...], preferred_element_type=jnp.float32) + bh[...]


def _row_spec(w):
    return pl.BlockSpec((BN, w), lambda i: (i, 0))


def _cnt_spec():
    return pl.BlockSpec((NW, BN), lambda i: (0, i))


def _full_spec(r, cdim):
    return pl.BlockSpec((r, cdim), lambda i: (0, 0))


_layer1 = pl.pallas_call(
    _layer1_body,
    grid=(NPAD // BN,),
    in_specs=[_row_spec(D), _row_spec(D), _cnt_spec(),
              _row_spec(D), _full_spec(D, D), _full_spec(D, D), _full_spec(1, D)],
    out_specs=_row_spec(D),
    out_shape=jax.ShapeDtypeStruct((NPAD, D), jnp.float32),
)

_layer2 = pl.pallas_call(
    _layer2_body,
    grid=(NPAD // BN,),
    in_specs=[_row_spec(D), _row_spec(D), _cnt_spec(),
              _row_spec(D), _full_spec(D, D), _full_spec(D, D), _full_spec(1, D),
              _full_spec(D, D), _full_spec(1, D)],
    out_specs=_row_spec(D),
    out_shape=jax.ShapeDtypeStruct((NPAD, D), jnp.float32),
)


def kernel(x, edge_index, W1_l, b1, W1_r, W2_l, b2, W2_r, W_head, b_head):
    src = edge_index[0]
    dst = edge_index[1]
    pad = EPAD - E
    src_p = jnp.concatenate([src, jnp.zeros((pad,), jnp.int32)])
    dst_p = jnp.concatenate([dst, jnp.full((pad,), N, jnp.int32)])
    src_2d = src_p.reshape(EPAD // CHUNK, CHUNK)
    dst_2d = dst_p.reshape(EPAD // CHUNK, CHUNK)
    x_p = jnp.zeros((NPAD, D), jnp.float32).at[:N].set(x)

    cnts = _degree(dst_p)
    sums1 = _seg_sum(src_2d, dst_2d, x_p)
    h1 = _layer1(sums1[0], sums1[1], cnts, x_p, W1_l, W1_r, b1.reshape(1, D))
    sums2 = _seg_sum(src_2d, dst_2d, h1)
    wh = jnp.zeros((D, D), jnp.float32).at[:, :3].set(W_head)
    bh = jnp.zeros((1, D), jnp.float32).at[0, :3].set(b_head)
    out = _layer2(sums2[0], sums2[1], cnts, h1, W2_l, W2_r, b2.reshape(1, D),
                  wh, bh)
    return out[:N, :3]


# speedup vs baseline: 1.3904x; 1.3904x over previous
import functools

import jax
import jax.numpy as jnp
from jax import lax
from jax.experimental import pallas as pl
from jax.experimental.pallas import tpu as pltpu
from jax.experimental.pallas import tpu_sc as plsc

N = 10000
E = 320000
D = 128
NC = 2
NS = 16
NW = NC * NS
CHUNK = 128
PT = ((E + NW - 1) // NW + CHUNK - 1) // CHUNK * CHUNK
EPAD = PT * NW
NCHUNK = PT // CHUNK
TOTCH = EPAD // CHUNK
NCH0 = 97
NCH1 = TOTCH // NS - NCH0
NPAD = 10240
STRIPE = NPAD // NS

_mesh = plsc.VectorSubcoreMesh(core_axis_name="c", subcore_axis_name="s",
                               num_cores=NC, num_subcores=NS)


def _zero_fill(buf, nrows, ncols):
    z16 = jnp.zeros((16,), jnp.float32)

    def fill(i, _):
        buf[i // (ncols // 16), pl.ds((i % (ncols // 16)) * 16, 16)] = z16
        return 0

    lax.fori_loop(0, nrows * (ncols // 16), fill, 0)


def _seg_core(src_hbm, dst_hbm, feat_hbm, out_sum, sum_sh, sidx, didx, rows):
    c = lax.axis_index("c")
    s = lax.axis_index("s")

    _zero_fill(rows, CHUNK, D)
    row0 = s * STRIPE

    def zcopy(j, _):
        pltpu.sync_copy(rows, sum_sh.at[pl.ds(row0 + j * CHUNK, CHUNK), :])
        return 0

    lax.fori_loop(0, STRIPE // CHUNK, zcopy, 0)
    plsc.subcore_barrier()

    nch = jnp.where(c == 0, NCH0, NCH1)
    cbase = jnp.where(c == 0, s * NCH0, NS * NCH0 + s * NCH1)

    def echunk(i, _):
        base = (cbase + i) * CHUNK
        pltpu.sync_copy(src_hbm.at[pl.ds(base, CHUNK)], sidx)
        pltpu.sync_copy(dst_hbm.at[pl.ds(base, CHUNK)], didx)
        pltpu.sync_copy(feat_hbm.at[sidx], rows)
        pltpu.sync_copy(rows, sum_sh.at[didx], add=True)
        return 0

    lax.fori_loop(0, nch, echunk, 0)
    plsc.subcore_barrier()

    pltpu.sync_copy(sum_sh.at[pl.ds(row0, STRIPE), :],
                    out_sum.at[c, pl.ds(row0, STRIPE), :])


@functools.partial(
    pl.kernel,
    out_type=jax.ShapeDtypeStruct((NW, NPAD), jnp.float32),
    mesh=_mesh,
    scratch_types=dict(
        didx=pltpu.VMEM((CHUNK,), jnp.int32),
        cntv=pltpu.VMEM((NPAD,), jnp.float32),
    ),
    compiler_params=pltpu.CompilerParams(needs_layout_passes=False),
)
def _degree(dst_hbm, out_cnt, didx, cntv):
    c = lax.axis_index("c")
    s = lax.axis_index("s")
    z16 = jnp.zeros((16,), jnp.float32)

    def czf(i, _):
        cntv[pl.ds(i * 16, 16)] = z16
        return 0

    lax.fori_loop(0, NPAD // 16, czf, 0)
    g = c * NS + s
    ebase = g * PT
    one16 = jnp.ones((16,), jnp.float32)

    def echunk(i, _):
        pltpu.sync_copy(dst_hbm.at[pl.ds(ebase + i * CHUNK, CHUNK)], didx)

        def cadd(j, _):
            v = didx[pl.ds(j * 16, 16)]
            plsc.addupdate_scatter(cntv, [v], one16)
            return 0

        lax.fori_loop(0, CHUNK // 16, cadd, 0)
        return 0

    lax.fori_loop(0, NCHUNK, echunk, 0)
    pltpu.sync_copy(cntv, out_cnt.at[g])


@functools.partial(
    pl.kernel,
    out_type=jax.ShapeDtypeStruct((NC, NPAD, D), jnp.float32),
    mesh=_mesh,
    scratch_types=dict(
        sidx=pltpu.VMEM((CHUNK,), jnp.int32),
        didx=pltpu.VMEM((CHUNK,), jnp.int32),
        rows=pltpu.VMEM((CHUNK, D), jnp.float32),
        sum_sh=pltpu.VMEM_SHARED((NPAD, D), jnp.float32),
    ),
)
def _seg_sum(src_hbm, dst_hbm, feat_hbm, out_sum,
             sidx, didx, rows, sum_sh):
    _seg_core(src_hbm, dst_hbm, feat_hbm, out_sum, sum_sh, sidx, didx, rows)


BN = 1024


def _layer1_body(p0, p1, cn, x, wl, wr, b, out):
    cnt = jnp.maximum(jnp.sum(cn[...], axis=0), 1.0)
    mean = (p0[...] + p1[...]) / cnt[:, None]
    acc = jnp.dot(mean, wl[...], preferred_element_type=jnp.float32)
    acc = acc + jnp.dot(x[...], wr[...], preferred_element_type=jnp.float32)
    out[...] = jnp.maximum(acc + b[...], 0.0)


def _layer2_body(q0, q1, cn, h, wl, wr, b, wh, bh, out):
    cnt = jnp.maximum(jnp.sum(cn[...], axis=0), 1.0)
    mean = (q0[...] + q1[...]) / cnt[:, None]
    acc = jnp.dot(mean, wl[...], preferred_element_type=jnp.float32)
    acc = acc + jnp.dot(h[...], wr[...], preferred_element_type=jnp.float32)
    h2 = jnp.maximum(acc + b[...], 0.0)
    out[...] = jnp.dot(h2, wh[...], preferred_element_type=jnp.float32) + bh[...]


def _row_spec(w):
    return pl.BlockSpec((BN, w), lambda i: (i, 0))


def _cnt_spec():
    return pl.BlockSpec((NW, BN), lambda i: (0, i))


def _full_spec(r, cdim):
    return pl.BlockSpec((r, cdim), lambda i: (0, 0))


_layer1 = pl.pallas_call(
    _layer1_body,
    grid=(NPAD // BN,),
    in_specs=[_row_spec(D), _row_spec(D), _cnt_spec(),
              _row_spec(D), _full_spec(D, D), _full_spec(D, D), _full_spec(1, D)],
    out_specs=_row_spec(D),
    out_shape=jax.ShapeDtypeStruct((NPAD, D), jnp.float32),
)

_layer2 = pl.pallas_call(
    _layer2_body,
    grid=(NPAD // BN,),
    in_specs=[_row_spec(D), _row_spec(D), _cnt_spec(),
              _row_spec(D), _full_spec(D, D), _full_spec(D, D), _full_spec(1, D),
              _full_spec(D, D), _full_spec(1, D)],
    out_specs=_row_spec(D),
    out_shape=jax.ShapeDtypeStruct((NPAD, D), jnp.float32),
)


def kernel(x, edge_index, W1_l, b1, W1_r, W2_l, b2, W2_r, W_head, b_head):
    src = edge_index[0]
    dst = edge_index[1]
    pad = EPAD - E
    src_p = jnp.concatenate([src, jnp.zeros((pad,), jnp.int32)])
    dst_p = jnp.concatenate([dst, jnp.full((pad,), N, jnp.int32)])
    x_p = jnp.zeros((NPAD, D), jnp.float32).at[:N].set(x)

    cnts = _degree(dst_p)
    sums1 = _seg_sum(src_p, dst_p, x_p)
    h1 = _layer1(sums1[0], sums1[1], cnts, x_p, W1_l, W1_r, b1.reshape(1, D))
    sums2 = _seg_sum(src_p, dst_p, h1)
    wh = jnp.zeros((D, D), jnp.float32).at[:, :3].set(W_head)
    bh = jnp.zeros((1, D), jnp.float32).at[0, :3].set(b_head)
    out = _layer2(sums2[0], sums2[1], cnts, h1, W2_l, W2_r, b2.reshape(1, D),
                  wh, bh)
    return out[:N, :3]


# speedup vs baseline: 1.5240x; 1.0960x over previous
import functools

import jax
import jax.numpy as jnp
from jax import lax
from jax.experimental import pallas as pl
from jax.experimental.pallas import tpu as pltpu
from jax.experimental.pallas import tpu_sc as plsc

N = 10000
E = 320000
D = 128
NC = 2
NS = 16
NW = NC * NS
CHUNK = 128
PT = ((E + NW - 1) // NW + CHUNK - 1) // CHUNK * CHUNK
EPAD = PT * NW
NCHUNK = PT // CHUNK
TOTCH = EPAD // CHUNK
NCH0 = 104
NCH1 = TOTCH // NS - NCH0
NPAD = 10240
STRIPE = NPAD // NS

_mesh = plsc.VectorSubcoreMesh(core_axis_name="c", subcore_axis_name="s",
                               num_cores=NC, num_subcores=NS)


def _zero_fill(buf, nrows, ncols):
    z16 = jnp.zeros((16,), jnp.float32)

    def fill(i, _):
        buf[i // (ncols // 16), pl.ds((i % (ncols // 16)) * 16, 16)] = z16
        return 0

    lax.fori_loop(0, nrows * (ncols // 16), fill, 0)


def _seg_core(src_hbm, dst_hbm, feat_hbm, out_sum, sum_sh, sidx, didx, rows,
              zrows_hbm=None, cnt_pack=None):
    c = lax.axis_index("c")
    s = lax.axis_index("s")

    if zrows_hbm is None:
        _zero_fill(rows, CHUNK, D)
    else:
        pltpu.sync_copy(zrows_hbm, rows)
    if cnt_pack is not None:
        out_cnt, cntv = cnt_pack
        z16 = jnp.zeros((16,), jnp.float32)

        def czf(i, _):
            cntv[pl.ds(i * 16, 16)] = z16
            return 0

        lax.fori_loop(0, NPAD // 16, czf, 0)

    row0 = s * STRIPE

    def zcopy(j, _):
        pltpu.sync_copy(rows, sum_sh.at[pl.ds(row0 + j * CHUNK, CHUNK), :])
        return 0

    lax.fori_loop(0, STRIPE // CHUNK, zcopy, 0)
    plsc.subcore_barrier()

    nch = jnp.where(c == 0, NCH0, NCH1)
    cbase = jnp.where(c == 0, s * NCH0, NS * NCH0 + s * NCH1)
    one16 = jnp.ones((16,), jnp.float32)

    def echunk(i, _):
        base = (cbase + i) * CHUNK
        pltpu.sync_copy(src_hbm.at[pl.ds(base, CHUNK)], sidx)
        pltpu.sync_copy(dst_hbm.at[pl.ds(base, CHUNK)], didx)
        pltpu.sync_copy(feat_hbm.at[sidx], rows)
        pltpu.sync_copy(rows, sum_sh.at[didx], add=True)
        if cnt_pack is not None:
            def cadd(j, _):
                v = didx[pl.ds(j * 16, 16)]
                plsc.addupdate_scatter(cntv, [v], one16)
                return 0

            lax.fori_loop(0, CHUNK // 16, cadd, 0)
        return 0

    lax.fori_loop(0, nch, echunk, 0)
    plsc.subcore_barrier()

    pltpu.sync_copy(sum_sh.at[pl.ds(row0, STRIPE), :],
                    out_sum.at[c, pl.ds(row0, STRIPE), :])
    if cnt_pack is not None:
        pltpu.sync_copy(cntv, out_cnt.at[c * NS + s])


@functools.partial(
    pl.kernel,
    out_type=(jax.ShapeDtypeStruct((NC, NPAD, D), jnp.float32),
              jax.ShapeDtypeStruct((NW, NPAD), jnp.float32)),
    mesh=_mesh,
    scratch_types=dict(
        sidx=pltpu.VMEM((CHUNK,), jnp.int32),
        didx=pltpu.VMEM((CHUNK,), jnp.int32),
        rows=pltpu.VMEM((CHUNK, D), jnp.float32),
        cntv=pltpu.VMEM((NPAD,), jnp.float32),
        sum_sh=pltpu.VMEM_SHARED((NPAD, D), jnp.float32),
    ),
    compiler_params=pltpu.CompilerParams(needs_layout_passes=False),
)
def _seg_sum_deg(src_hbm, dst_hbm, feat_hbm, zrows_hbm, out_sum, out_cnt,
                 sidx, didx, rows, cntv, sum_sh):
    _seg_core(src_hbm, dst_hbm, feat_hbm, out_sum, sum_sh, sidx, didx, rows,
              zrows_hbm=zrows_hbm, cnt_pack=(out_cnt, cntv))


@functools.partial(
    pl.kernel,
    out_type=jax.ShapeDtypeStruct((NC, NPAD, D), jnp.float32),
    mesh=_mesh,
    scratch_types=dict(
        sidx=pltpu.VMEM((CHUNK,), jnp.int32),
        didx=pltpu.VMEM((CHUNK,), jnp.int32),
        rows=pltpu.VMEM((CHUNK, D), jnp.float32),
        sum_sh=pltpu.VMEM_SHARED((NPAD, D), jnp.float32),
    ),
)
def _seg_sum(src_hbm, dst_hbm, feat_hbm, out_sum,
             sidx, didx, rows, sum_sh):
    _seg_core(src_hbm, dst_hbm, feat_hbm, out_sum, sum_sh, sidx, didx, rows)


BN = 1000
CB = 2560


def _cntred_body(cn, out):
    out[...] = (1.0 / jnp.maximum(jnp.sum(cn[...], axis=0), 1.0))[:, None]


def _layer1_body(p0, p1, rcp, x, wl, wr, b, out):
    mean = (p0[...] + p1[...]) * rcp[...]
    acc = jnp.dot(mean, wl[...], preferred_element_type=jnp.float32)
    acc = acc + jnp.dot(x[...], wr[...], preferred_element_type=jnp.float32)
    out[...] = jnp.maximum(acc + b[...], 0.0)


def _layer2_body(q0, q1, rcp, h, wl, wr, b, wh, bh, out):
    mean = (q0[...] + q1[...]) * rcp[...]
    acc = jnp.dot(mean, wl[...], preferred_element_type=jnp.float32)
    acc = acc + jnp.dot(h[...], wr[...], preferred_element_type=jnp.float32)
    h2 = jnp.maximum(acc + b[...], 0.0)
    out[...] = jnp.dot(h2, wh[...], preferred_element_type=jnp.float32) + bh[...]


def _row_spec(w):
    return pl.BlockSpec((BN, w), lambda i: (i, 0))


def _rcp_spec():
    return pl.BlockSpec((BN, 1), lambda i: (i, 0))


def _full_spec(r, cdim):
    return pl.BlockSpec((r, cdim), lambda i: (0, 0))


_cntred = pl.pallas_call(
    _cntred_body,
    grid=(NPAD // CB,),
    in_specs=[pl.BlockSpec((NW, CB), lambda i: (0, i))],
    out_specs=pl.BlockSpec((CB, 1), lambda i: (i, 0)),
    out_shape=jax.ShapeDtypeStruct((NPAD, 1), jnp.float32),
)


_layer1 = pl.pallas_call(
    _layer1_body,
    grid=(N // BN,),
    in_specs=[_row_spec(D), _row_spec(D), _rcp_spec(),
              _row_spec(D), _full_spec(D, D), _full_spec(D, D), _full_spec(1, D)],
    out_specs=_row_spec(D),
    out_shape=jax.ShapeDtypeStruct((N, D), jnp.float32),
)

_layer2 = pl.pallas_call(
    _layer2_body,
    grid=(N // BN,),
    in_specs=[_row_spec(D), _row_spec(D), _rcp_spec(),
              _row_spec(D), _full_spec(D, D), _full_spec(D, D), _full_spec(1, D),
              _full_spec(D, D), _full_spec(1, D)],
    out_specs=_row_spec(D),
    out_shape=jax.ShapeDtypeStruct((N, D), jnp.float32),
)


def kernel(x, edge_index, W1_l, b1, W1_r, W2_l, b2, W2_r, W_head, b_head):
    src = edge_index[0]
    dst = edge_index[1]
    pad = EPAD - E
    src_p = jnp.concatenate([src, jnp.zeros((pad,), jnp.int32)])
    dst_p = jnp.concatenate([dst, jnp.full((pad,), N, jnp.int32)])
    zrows = jnp.zeros((CHUNK, D), jnp.float32)

    sums1, cnts = _seg_sum_deg(src_p, dst_p, x, zrows)
    rcp = _cntred(cnts)
    h1 = _layer1(sums1[0], sums1[1], rcp, x, W1_l, W1_r, b1.reshape(1, D))
    sums2 = _seg_sum(src_p, dst_p, h1)
    wh = jnp.zeros((D, D), jnp.float32).at[:, :3].set(W_head)
    bh = jnp.zeros((1, D), jnp.float32).at[0, :3].set(b_head)
    out = _layer2(sums2[0], sums2[1], rcp, h1, W2_l, W2_r, b2.reshape(1, D),
                  wh, bh)
    return out[:, :3]


# speedup vs baseline: 1.6987x; 1.1147x over previous
import functools

import jax
import jax.numpy as jnp
from jax import lax
from jax.experimental import pallas as pl
from jax.experimental.pallas import tpu as pltpu
from jax.experimental.pallas import tpu_sc as plsc

N = 10000
E = 320000
D = 128
NC = 2
NS = 16
NW = NC * NS
CHUNK = 128
PT = ((E + NW - 1) // NW + CHUNK - 1) // CHUNK * CHUNK
EPAD = PT * NW
NCHUNK = PT // CHUNK
TOTCH = EPAD // CHUNK
NCH0 = 104
NCH1 = TOTCH // NS - NCH0
NPAD = 10240
STRIPE = NPAD // NS

_mesh = plsc.VectorSubcoreMesh(core_axis_name="c", subcore_axis_name="s",
                               num_cores=NC, num_subcores=NS)


def _zero_fill(buf, nrows, ncols):
    z16 = jnp.zeros((16,), jnp.float32)

    def fill(i, _):
        buf[i // (ncols // 16), pl.ds((i % (ncols // 16)) * 16, 16)] = z16
        return 0

    lax.fori_loop(0, nrows * (ncols // 16), fill, 0)


def _seg_core(src_hbm, dst_hbm, feat_hbm, out_sum, sum_sh, sidx, didx, rows,
              zrows_hbm=None, cnt_pack=None):
    c = lax.axis_index("c")
    s = lax.axis_index("s")

    if zrows_hbm is None:
        _zero_fill(rows, CHUNK, D)
    else:
        pltpu.sync_copy(zrows_hbm, rows)
    if cnt_pack is not None:
        out_cnt, cntv = cnt_pack
        z16 = jnp.zeros((16,), jnp.float32)

        def czf(i, _):
            cntv[pl.ds(i * 16, 16)] = z16
            return 0

        lax.fori_loop(0, NPAD // 16, czf, 0)

    row0 = s * STRIPE

    def zcopy(j, _):
        pltpu.sync_copy(rows, sum_sh.at[pl.ds(row0 + j * CHUNK, CHUNK), :])
        return 0

    lax.fori_loop(0, STRIPE // CHUNK, zcopy, 0)
    plsc.subcore_barrier()

    nch = jnp.where(c == 0, NCH0, NCH1)
    cbase = jnp.where(c == 0, s * NCH0, NS * NCH0 + s * NCH1)
    one16 = jnp.ones((16,), jnp.float32)

    def echunk(i, _):
        base = (cbase + i) * CHUNK
        pltpu.sync_copy(src_hbm.at[pl.ds(base, CHUNK)], sidx)
        pltpu.sync_copy(dst_hbm.at[pl.ds(base, CHUNK)], didx)
        pltpu.sync_copy(feat_hbm.at[sidx], rows)
        pltpu.sync_copy(rows, sum_sh.at[didx], add=True)
        if cnt_pack is not None:
            def cadd(j, _):
                v = didx[pl.ds(j * 16, 16)]
                plsc.addupdate_scatter(cntv, [v], one16)
                return 0

            lax.fori_loop(0, CHUNK // 16, cadd, 0)
        return 0

    lax.fori_loop(0, nch, echunk, 0)
    plsc.subcore_barrier()

    pltpu.sync_copy(sum_sh.at[pl.ds(row0, STRIPE), :],
                    out_sum.at[c, pl.ds(row0, STRIPE), :])
    if cnt_pack is not None:
        pltpu.sync_copy(cntv, out_cnt.at[c * NS + s])


@functools.partial(
    pl.kernel,
    out_type=(jax.ShapeDtypeStruct((NC, NPAD, D), jnp.float32),
              jax.ShapeDtypeStruct((NW, NPAD), jnp.float32)),
    mesh=_mesh,
    scratch_types=dict(
        sidx=pltpu.VMEM((CHUNK,), jnp.int32),
        didx=pltpu.VMEM((CHUNK,), jnp.int32),
        rows=pltpu.VMEM((CHUNK, D), jnp.float32),
        cntv=pltpu.VMEM((NPAD,), jnp.float32),
        sum_sh=pltpu.VMEM_SHARED((NPAD, D), jnp.float32),
    ),
    compiler_params=pltpu.CompilerParams(needs_layout_passes=False),
)
def _seg_sum_deg(src_hbm, dst_hbm, feat_hbm, zrows_hbm, out_sum, out_cnt,
                 sidx, didx, rows, cntv, sum_sh):
    _seg_core(src_hbm, dst_hbm, feat_hbm, out_sum, sum_sh, sidx, didx, rows,
              zrows_hbm=zrows_hbm, cnt_pack=(out_cnt, cntv))


@functools.partial(
    pl.kernel,
    out_type=jax.ShapeDtypeStruct((NC, NPAD, D), jnp.float32),
    mesh=_mesh,
    scratch_types=dict(
        sidx0=pltpu.VMEM((CHUNK,), jnp.int32),
        sidx1=pltpu.VMEM((CHUNK,), jnp.int32),
        didx0=pltpu.VMEM((CHUNK,), jnp.int32),
        didx1=pltpu.VMEM((CHUNK,), jnp.int32),
        rows0=pltpu.VMEM((CHUNK, D), jnp.float32),
        rows1=pltpu.VMEM((CHUNK, D), jnp.float32),
        sum_sh=pltpu.VMEM_SHARED((NPAD, D), jnp.float32),
        gsem0=pltpu.SemaphoreType.DMA,
        gsem1=pltpu.SemaphoreType.DMA,
    ),
)
def _seg_sum(src_hbm, dst_hbm, feat_hbm, out_sum,
             sidx0, sidx1, didx0, didx1, rows0, rows1, sum_sh, gsem0, gsem1):
    c = lax.axis_index("c")
    s = lax.axis_index("s")

    _zero_fill(rows0, CHUNK, D)
    row0 = s * STRIPE

    def zcopy(j, _):
        pltpu.sync_copy(rows0, sum_sh.at[pl.ds(row0 + j * CHUNK, CHUNK), :])
        return 0

    lax.fori_loop(0, STRIPE // CHUNK, zcopy, 0)
    plsc.subcore_barrier()

    nch = jnp.where(c == 0, NCH0, NCH1)
    cbase = jnp.where(c == 0, s * NCH0, NS * NCH0 + s * NCH1)
    sidx = (sidx0, sidx1)
    didx = (didx0, didx1)
    rows = (rows0, rows1)
    sems = (gsem0, gsem1)

    def load_and_fire(i, par):
        base = (cbase + i) * CHUNK
        pltpu.sync_copy(src_hbm.at[pl.ds(base, CHUNK)], sidx[par])
        pltpu.sync_copy(dst_hbm.at[pl.ds(base, CHUNK)], didx[par])
        return pltpu.async_copy(feat_hbm.at[sidx[par]], rows[par], sems[par])

    d0 = load_and_fire(0, 0)

    def epair(p, _):
        i0 = 2 * p

        @pl.when(i0 + 1 < nch)
        def _():
            load_and_fire(i0 + 1, 1)

        pltpu.make_async_copy(feat_hbm.at[sidx0], rows0, gsem0).wait()
        pltpu.sync_copy(rows0, sum_sh.at[didx0], add=True)

        @pl.when(i0 + 1 < nch)
        def _():
            @pl.when(i0 + 2 < nch)
            def _():
                load_and_fire(i0 + 2, 0)

            pltpu.make_async_copy(feat_hbm.at[sidx1], rows1, gsem1).wait()
            pltpu.sync_copy(rows1, sum_sh.at[didx1], add=True)

        return 0

    del d0
    lax.fori_loop(0, (jnp.asarray(nch) + 1) // 2, epair, 0)
    plsc.subcore_barrier()

    pltpu.sync_copy(sum_sh.at[pl.ds(row0, STRIPE), :],
                    out_sum.at[c, pl.ds(row0, STRIPE), :])


BN = 1000
CB = 2560


def _cntred_body(cn, out):
    out[...] = (1.0 / jnp.maximum(jnp.sum(cn[...], axis=0), 1.0))[:, None]


def _layer1_body(p0, p1, rcp, x, wl, wr, b, out):
    mean = (p0[...] + p1[...]) * rcp[...]
    acc = jnp.dot(mean, wl[...], preferred_element_type=jnp.float32)
    acc = acc + jnp.dot(x[...], wr[...], preferred_element_type=jnp.float32)
    out[...] = jnp.maximum(acc + b[...], 0.0)


def _layer2_body(q0, q1, rcp, h, wl, wr, b, wh, bh, out):
    mean = (q0[...] + q1[...]) * rcp[...]
    acc = jnp.dot(mean, wl[...], preferred_element_type=jnp.float32)
    acc = acc + jnp.dot(h[...], wr[...], preferred_element_type=jnp.float32)
    h2 = jnp.maximum(acc + b[...], 0.0)
    out[...] = jnp.dot(h2, wh[...], preferred_element_type=jnp.float32) + bh[...]


def _row_spec(w):
    return pl.BlockSpec((BN, w), lambda i: (i, 0))


def _rcp_spec():
    return pl.BlockSpec((BN, 1), lambda i: (i, 0))


def _full_spec(r, cdim):
    return pl.BlockSpec((r, cdim), lambda i: (0, 0))


_cntred = pl.pallas_call(
    _cntred_body,
    grid=(NPAD // CB,),
    in_specs=[pl.BlockSpec((NW, CB), lambda i: (0, i))],
    out_specs=pl.BlockSpec((CB, 1), lambda i: (i, 0)),
    out_shape=jax.ShapeDtypeStruct((NPAD, 1), jnp.float32),
)


_layer1 = pl.pallas_call(
    _layer1_body,
    grid=(N // BN,),
    in_specs=[_row_spec(D), _row_spec(D), _rcp_spec(),
              _row_spec(D), _full_spec(D, D), _full_spec(D, D), _full_spec(1, D)],
    out_specs=_row_spec(D),
    out_shape=jax.ShapeDtypeStruct((N, D), jnp.float32),
)

_layer2 = pl.pallas_call(
    _layer2_body,
    grid=(N // BN,),
    in_specs=[_row_spec(D), _row_spec(D), _rcp_spec(),
              _row_spec(D), _full_spec(D, D), _full_spec(D, D), _full_spec(1, D),
              _full_spec(D, D), _full_spec(1, D)],
    out_specs=_row_spec(D),
    out_shape=jax.ShapeDtypeStruct((N, D), jnp.float32),
)


def kernel(x, edge_index, W1_l, b1, W1_r, W2_l, b2, W2_r, W_head, b_head):
    src = edge_index[0]
    dst = edge_index[1]
    pad = EPAD - E
    src_p = jnp.concatenate([src, jnp.zeros((pad,), jnp.int32)])
    dst_p = jnp.concatenate([dst, jnp.full((pad,), N, jnp.int32)])
    zrows = jnp.zeros((CHUNK, D), jnp.float32)

    sums1, cnts = _seg_sum_deg(src_p, dst_p, x, zrows)
    rcp = _cntred(cnts)
    h1 = _layer1(sums1[0], sums1[1], rcp, x, W1_l, W1_r, b1.reshape(1, D))
    sums2 = _seg_sum(src_p, dst_p, h1)
    wh = jnp.zeros((D, D), jnp.float32).at[:, :3].set(W_head)
    bh = jnp.zeros((1, D), jnp.float32).at[0, :3].set(b_head)
    out = _layer2(sums2[0], sums2[1], rcp, h1, W2_l, W2_r, b2.reshape(1, D),
                  wh, bh)
    return out[:, :3]


# speedup vs baseline: 1.9140x; 1.1267x over previous
import functools

import jax
import jax.numpy as jnp
from jax import lax
from jax.experimental import pallas as pl
from jax.experimental.pallas import tpu as pltpu
from jax.experimental.pallas import tpu_sc as plsc

N = 10000
E = 320000
D = 128
NC = 2
NS = 16
NW = NC * NS
CHUNK = 128
PT = ((E + NW - 1) // NW + CHUNK - 1) // CHUNK * CHUNK
EPAD = PT * NW
NCHUNK = PT // CHUNK
TOTCH = EPAD // CHUNK
NCH0 = 104
NCH1 = TOTCH // NS - NCH0
NPAD = 10240
STRIPE = NPAD // NS

_mesh = plsc.VectorSubcoreMesh(core_axis_name="c", subcore_axis_name="s",
                               num_cores=NC, num_subcores=NS)


def _zero_fill(buf, nrows, ncols):
    z16 = jnp.zeros((16,), jnp.float32)

    def fill(i, _):
        buf[i // (ncols // 16), pl.ds((i % (ncols // 16)) * 16, 16)] = z16
        return 0

    lax.fori_loop(0, nrows * (ncols // 16), fill, 0)


def _seg_pipe(src_hbm, dst_hbm, feat_hbm, out_sum, sum_sh, sidx, didx, rows,
              sems, zrows_hbm=None, cnt_pack=None):
    c = lax.axis_index("c")
    s = lax.axis_index("s")

    if zrows_hbm is None:
        _zero_fill(rows[0], CHUNK, D)
    else:
        pltpu.sync_copy(zrows_hbm, rows[0])
    if cnt_pack is not None:
        out_cnt, cntv = cnt_pack
        z16 = jnp.zeros((16,), jnp.float32)

        def czf(i, _):
            cntv[pl.ds(i * 16, 16)] = z16
            return 0

        lax.fori_loop(0, NPAD // 16, czf, 0)

    row0 = s * STRIPE

    def zcopy(j, _):
        pltpu.sync_copy(rows[0], sum_sh.at[pl.ds(row0 + j * CHUNK, CHUNK), :])
        return 0

    lax.fori_loop(0, STRIPE // CHUNK, zcopy, 0)
    plsc.subcore_barrier()

    nch = jnp.where(c == 0, NCH0, NCH1)
    cbase = jnp.where(c == 0, s * NCH0, NS * NCH0 + s * NCH1)
    one16 = jnp.ones((16,), jnp.float32)

    def load_and_fire(i, par):
        base = (cbase + i) * CHUNK
        pltpu.sync_copy(src_hbm.at[pl.ds(base, CHUNK)], sidx[par])
        pltpu.sync_copy(dst_hbm.at[pl.ds(base, CHUNK)], didx[par])
        return pltpu.async_copy(feat_hbm.at[sidx[par]], rows[par], sems[par])

    def drain_scatter(par):
        pltpu.make_async_copy(feat_hbm.at[sidx[par]], rows[par],
                              sems[par]).wait()
        pltpu.sync_copy(rows[par], sum_sh.at[didx[par]], add=True)
        if cnt_pack is not None:
            def cadd(j, _):
                v = didx[par][pl.ds(j * 16, 16)]
                plsc.addupdate_scatter(cntv, [v], one16)
                return 0

            lax.fori_loop(0, CHUNK // 16, cadd, 0)

    load_and_fire(0, 0)

    def epair(p, _):
        i0 = 2 * p

        @pl.when(i0 + 1 < nch)
        def _():
            load_and_fire(i0 + 1, 1)

        drain_scatter(0)

        @pl.when(i0 + 1 < nch)
        def _():
            @pl.when(i0 + 2 < nch)
            def _():
                load_and_fire(i0 + 2, 0)

            drain_scatter(1)

        return 0

    lax.fori_loop(0, (nch + 1) // 2, epair, 0)
    plsc.subcore_barrier()

    pltpu.sync_copy(sum_sh.at[pl.ds(row0, STRIPE), :],
                    out_sum.at[c, pl.ds(row0, STRIPE), :])
    if cnt_pack is not None:
        pltpu.sync_copy(cntv, out_cnt.at[c * NS + s])


@functools.partial(
    pl.kernel,
    out_type=(jax.ShapeDtypeStruct((NC, NPAD, D), jnp.float32),
              jax.ShapeDtypeStruct((NW, NPAD), jnp.float32)),
    mesh=_mesh,
    scratch_types=dict(
        sidx0=pltpu.VMEM((CHUNK,), jnp.int32),
        sidx1=pltpu.VMEM((CHUNK,), jnp.int32),
        didx0=pltpu.VMEM((CHUNK,), jnp.int32),
        didx1=pltpu.VMEM((CHUNK,), jnp.int32),
        rows0=pltpu.VMEM((CHUNK, D), jnp.float32),
        rows1=pltpu.VMEM((CHUNK, D), jnp.float32),
        cntv=pltpu.VMEM((NPAD,), jnp.float32),
        sum_sh=pltpu.VMEM_SHARED((NPAD, D), jnp.float32),
        gsem0=pltpu.SemaphoreType.DMA,
        gsem1=pltpu.SemaphoreType.DMA,
    ),
    compiler_params=pltpu.CompilerParams(needs_layout_passes=False),
)
def _seg_sum_deg(src_hbm, dst_hbm, feat_hbm, zrows_hbm, out_sum, out_cnt,
                 sidx0, sidx1, didx0, didx1, rows0, rows1, cntv, sum_sh,
                 gsem0, gsem1):
    _seg_pipe(src_hbm, dst_hbm, feat_hbm, out_sum, sum_sh,
              (sidx0, sidx1), (didx0, didx1), (rows0, rows1), (gsem0, gsem1),
              zrows_hbm=zrows_hbm, cnt_pack=(out_cnt, cntv))


@functools.partial(
    pl.kernel,
    out_type=jax.ShapeDtypeStruct((NC, NPAD, D), jnp.float32),
    mesh=_mesh,
    scratch_types=dict(
        sidx0=pltpu.VMEM((CHUNK,), jnp.int32),
        sidx1=pltpu.VMEM((CHUNK,), jnp.int32),
        didx0=pltpu.VMEM((CHUNK,), jnp.int32),
        didx1=pltpu.VMEM((CHUNK,), jnp.int32),
        rows0=pltpu.VMEM((CHUNK, D), jnp.float32),
        rows1=pltpu.VMEM((CHUNK, D), jnp.float32),
        sum_sh=pltpu.VMEM_SHARED((NPAD, D), jnp.float32),
        gsem0=pltpu.SemaphoreType.DMA,
        gsem1=pltpu.SemaphoreType.DMA,
    ),
)
def _seg_sum(src_hbm, dst_hbm, feat_hbm, out_sum,
             sidx0, sidx1, didx0, didx1, rows0, rows1, sum_sh, gsem0, gsem1):
    _seg_pipe(src_hbm, dst_hbm, feat_hbm, out_sum, sum_sh,
              (sidx0, sidx1), (didx0, didx1), (rows0, rows1), (gsem0, gsem1))


BN = 1000
CB = 2560


def _cntred_body(cn, out):
    out[...] = (1.0 / jnp.maximum(jnp.sum(cn[...], axis=0), 1.0))[:, None]


def _layer1_body(p0, p1, rcp, x, wl, wr, b, out):
    mean = (p0[...] + p1[...]) * rcp[...]
    acc = jnp.dot(mean, wl[...], preferred_element_type=jnp.float32)
    acc = acc + jnp.dot(x[...], wr[...], preferred_element_type=jnp.float32)
    out[...] = jnp.maximum(acc + b[...], 0.0)


def _layer2_body(q0, q1, rcp, h, wl, wr, b, wh, bh, out):
    mean = (q0[...] + q1[...]) * rcp[...]
    acc = jnp.dot(mean, wl[...], preferred_element_type=jnp.float32)
    acc = acc + jnp.dot(h[...], wr[...], preferred_element_type=jnp.float32)
    h2 = jnp.maximum(acc + b[...], 0.0)
    out[...] = jnp.dot(h2, wh[...], preferred_element_type=jnp.float32) + bh[...]


def _row_spec(w):
    return pl.BlockSpec((BN, w), lambda i: (i, 0))


def _rcp_spec():
    return pl.BlockSpec((BN, 1), lambda i: (i, 0))


def _full_spec(r, cdim):
    return pl.BlockSpec((r, cdim), lambda i: (0, 0))


_cntred = pl.pallas_call(
    _cntred_body,
    grid=(NPAD // CB,),
    in_specs=[pl.BlockSpec((NW, CB), lambda i: (0, i))],
    out_specs=pl.BlockSpec((CB, 1), lambda i: (i, 0)),
    out_shape=jax.ShapeDtypeStruct((NPAD, 1), jnp.float32),
)


_layer1 = pl.pallas_call(
    _layer1_body,
    grid=(N // BN,),
    in_specs=[_row_spec(D), _row_spec(D), _rcp_spec(),
              _row_spec(D), _full_spec(D, D), _full_spec(D, D), _full_spec(1, D)],
    out_specs=_row_spec(D),
    out_shape=jax.ShapeDtypeStruct((N, D), jnp.float32),
)

_layer2 = pl.pallas_call(
    _layer2_body,
    grid=(N // BN,),
    in_specs=[_row_spec(D), _row_spec(D), _rcp_spec(),
              _row_spec(D), _full_spec(D, D), _full_spec(D, D), _full_spec(1, D),
              _full_spec(D, D), _full_spec(1, D)],
    out_specs=_row_spec(D),
    out_shape=jax.ShapeDtypeStruct((N, D), jnp.float32),
)


def kernel(x, edge_index, W1_l, b1, W1_r, W2_l, b2, W2_r, W_head, b_head):
    src = edge_index[0]
    dst = edge_index[1]
    pad = EPAD - E
    src_p = jnp.concatenate([src, jnp.zeros((pad,), jnp.int32)])
    dst_p = jnp.concatenate([dst, jnp.full((pad,), N, jnp.int32)])
    zrows = jnp.zeros((CHUNK, D), jnp.float32)

    sums1, cnts = _seg_sum_deg(src_p, dst_p, x, zrows)
    rcp = _cntred(cnts)
    h1 = _layer1(sums1[0], sums1[1], rcp, x, W1_l, W1_r, b1.reshape(1, D))
    sums2 = _seg_sum(src_p, dst_p, h1)
    wh = jnp.zeros((D, D), jnp.float32).at[:, :3].set(W_head)
    bh = jnp.zeros((1, D), jnp.float32).at[0, :3].set(b_head)
    out = _layer2(sums2[0], sums2[1], rcp, h1, W2_l, W2_r, b2.reshape(1, D),
                  wh, bh)
    return out[:, :3]


# speedup vs baseline: 2.0084x; 1.0494x over previous
import functools

import jax
import jax.numpy as jnp
from jax import lax
from jax.experimental import pallas as pl
from jax.experimental.pallas import tpu as pltpu
from jax.experimental.pallas import tpu_sc as plsc

N = 10000
E = 320000
D = 128
NC = 2
NS = 16
NW = NC * NS
CHUNK = 128
PT = ((E + NW - 1) // NW + CHUNK - 1) // CHUNK * CHUNK
EPAD = PT * NW
NCHUNK = PT // CHUNK
TOTCH = EPAD // CHUNK
NCH0 = 114
NCH1 = TOTCH // NS - NCH0
NPAD = 10240
STRIPE = NPAD // NS

_mesh = plsc.VectorSubcoreMesh(core_axis_name="c", subcore_axis_name="s",
                               num_cores=NC, num_subcores=NS)


def _zero_fill(buf, nrows, ncols):
    z16 = jnp.zeros((16,), jnp.float32)

    def fill(i, _):
        buf[i // (ncols // 16), pl.ds((i % (ncols // 16)) * 16, 16)] = z16
        return 0

    lax.fori_loop(0, nrows * (ncols // 16), fill, 0)


def _seg_pipe(src_hbm, dst_hbm, feat_hbm, out_sum, sum_sh, sidx, didx, rows,
              sems, zrows_hbm=None, cnt_pack=None):
    c = lax.axis_index("c")
    s = lax.axis_index("s")

    if zrows_hbm is None:
        _zero_fill(rows[0], CHUNK, D)
    else:
        pltpu.sync_copy(zrows_hbm, rows[0])
    if cnt_pack is not None:
        out_cnt, cntv = cnt_pack
        z16 = jnp.zeros((16,), jnp.float32)

        def czf(i, _):
            cntv[pl.ds(i * 16, 16)] = z16
            return 0

        lax.fori_loop(0, NPAD // 16, czf, 0)

    row0 = s * STRIPE

    def zcopy(j, _):
        pltpu.sync_copy(rows[0], sum_sh.at[pl.ds(row0 + j * CHUNK, CHUNK), :])
        return 0

    lax.fori_loop(0, STRIPE // CHUNK, zcopy, 0)
    plsc.subcore_barrier()

    nch = jnp.where(c == 0, NCH0, NCH1)
    cbase = jnp.where(c == 0, s * NCH0, NS * NCH0 + s * NCH1)
    one16 = jnp.ones((16,), jnp.float32)

    def load_and_fire(i, par):
        base = (cbase + i) * CHUNK
        pltpu.sync_copy(src_hbm.at[pl.ds(base, CHUNK)], sidx[par])
        pltpu.sync_copy(dst_hbm.at[pl.ds(base, CHUNK)], didx[par])
        return pltpu.async_copy(feat_hbm.at[sidx[par]], rows[par], sems[par])

    def drain_scatter(par):
        pltpu.make_async_copy(feat_hbm.at[sidx[par]], rows[par],
                              sems[par]).wait()
        pltpu.sync_copy(rows[par], sum_sh.at[didx[par]], add=True)
        if cnt_pack is not None:
            def cadd(j, _):
                v = didx[par][pl.ds(j * 16, 16)]
                plsc.addupdate_scatter(cntv, [v], one16)
                return 0

            lax.fori_loop(0, CHUNK // 16, cadd, 0)

    load_and_fire(0, 0)

    def epair(p, _):
        i0 = 2 * p

        @pl.when(i0 + 1 < nch)
        def _():
            load_and_fire(i0 + 1, 1)

        drain_scatter(0)

        @pl.when(i0 + 1 < nch)
        def _():
            @pl.when(i0 + 2 < nch)
            def _():
                load_and_fire(i0 + 2, 0)

            drain_scatter(1)

        return 0

    lax.fori_loop(0, (nch + 1) // 2, epair, 0)
    plsc.subcore_barrier()

    pltpu.sync_copy(sum_sh.at[pl.ds(row0, STRIPE), :],
                    out_sum.at[c, pl.ds(row0, STRIPE), :])
    if cnt_pack is not None:
        pltpu.sync_copy(cntv, out_cnt.at[c * NS + s])


@functools.partial(
    pl.kernel,
    out_type=(jax.ShapeDtypeStruct((NC, NPAD, D), jnp.float32),
              jax.ShapeDtypeStruct((NW, NPAD), jnp.float32)),
    mesh=_mesh,
    scratch_types=dict(
        sidx0=pltpu.VMEM((CHUNK,), jnp.int32),
        sidx1=pltpu.VMEM((CHUNK,), jnp.int32),
        didx0=pltpu.VMEM((CHUNK,), jnp.int32),
        didx1=pltpu.VMEM((CHUNK,), jnp.int32),
        rows0=pltpu.VMEM((CHUNK, D), jnp.float32),
        rows1=pltpu.VMEM((CHUNK, D), jnp.float32),
        cntv=pltpu.VMEM((NPAD,), jnp.float32),
        sum_sh=pltpu.VMEM_SHARED((NPAD, D), jnp.float32),
        gsem0=pltpu.SemaphoreType.DMA,
        gsem1=pltpu.SemaphoreType.DMA,
    ),
    compiler_params=pltpu.CompilerParams(needs_layout_passes=False),
)
def _seg_sum_deg(src_hbm, dst_hbm, feat_hbm, zrows_hbm, out_sum, out_cnt,
                 sidx0, sidx1, didx0, didx1, rows0, rows1, cntv, sum_sh,
                 gsem0, gsem1):
    _seg_pipe(src_hbm, dst_hbm, feat_hbm, out_sum, sum_sh,
              (sidx0, sidx1), (didx0, didx1), (rows0, rows1), (gsem0, gsem1),
              zrows_hbm=zrows_hbm, cnt_pack=(out_cnt, cntv))


@functools.partial(
    pl.kernel,
    out_type=jax.ShapeDtypeStruct((NC, NPAD, D), jnp.float32),
    mesh=_mesh,
    scratch_types=dict(
        sidx0=pltpu.VMEM((CHUNK,), jnp.int32),
        sidx1=pltpu.VMEM((CHUNK,), jnp.int32),
        didx0=pltpu.VMEM((CHUNK,), jnp.int32),
        didx1=pltpu.VMEM((CHUNK,), jnp.int32),
        rows0=pltpu.VMEM((CHUNK, D), jnp.float32),
        rows1=pltpu.VMEM((CHUNK, D), jnp.float32),
        sum_sh=pltpu.VMEM_SHARED((NPAD, D), jnp.float32),
        gsem0=pltpu.SemaphoreType.DMA,
        gsem1=pltpu.SemaphoreType.DMA,
    ),
)
def _seg_sum(src_hbm, dst_hbm, feat_hbm, out_sum,
             sidx0, sidx1, didx0, didx1, rows0, rows1, sum_sh, gsem0, gsem1):
    _seg_pipe(src_hbm, dst_hbm, feat_hbm, out_sum, sum_sh,
              (sidx0, sidx1), (didx0, didx1), (rows0, rows1), (gsem0, gsem1))


BN = 1000
CB = 2560


def _cntred_body(cn, out):
    out[...] = (1.0 / jnp.maximum(jnp.sum(cn[...], axis=0), 1.0))[:, None]


def _layer1_body(p0, p1, rcp, x, wl, wr, b, out):
    mean = (p0[...] + p1[...]) * rcp[...]
    acc = jnp.dot(mean, wl[...], preferred_element_type=jnp.float32)
    acc = acc + jnp.dot(x[...], wr[...], preferred_element_type=jnp.float32)
    out[...] = jnp.maximum(acc + b[...], 0.0)


def _layer2_body(q0, q1, rcp, h, wl, wr, b, wh, bh, out):
    mean = (q0[...] + q1[...]) * rcp[...]
    acc = jnp.dot(mean, wl[...], preferred_element_type=jnp.float32)
    acc = acc + jnp.dot(h[...], wr[...], preferred_element_type=jnp.float32)
    h2 = jnp.maximum(acc + b[...], 0.0)
    out[...] = jnp.dot(h2, wh[...], preferred_element_type=jnp.float32) + bh[...]


def _row_spec(w):
    return pl.BlockSpec((BN, w), lambda i: (i, 0))


def _rcp_spec():
    return pl.BlockSpec((BN, 1), lambda i: (i, 0))


def _full_spec(r, cdim):
    return pl.BlockSpec((r, cdim), lambda i: (0, 0))


_cntred = pl.pallas_call(
    _cntred_body,
    grid=(NPAD // CB,),
    in_specs=[pl.BlockSpec((NW, CB), lambda i: (0, i))],
    out_specs=pl.BlockSpec((CB, 1), lambda i: (i, 0)),
    out_shape=jax.ShapeDtypeStruct((NPAD, 1), jnp.float32),
)


_layer1 = pl.pallas_call(
    _layer1_body,
    grid=(N // BN,),
    in_specs=[_row_spec(D), _row_spec(D), _rcp_spec(),
              _row_spec(D), _full_spec(D, D), _full_spec(D, D), _full_spec(1, D)],
    out_specs=_row_spec(D),
    out_shape=jax.ShapeDtypeStruct((N, D), jnp.float32),
)

_layer2 = pl.pallas_call(
    _layer2_body,
    grid=(N // BN,),
    in_specs=[_row_spec(D), _row_spec(D), _rcp_spec(),
              _row_spec(D), _full_spec(D, D), _full_spec(D, D), _full_spec(1, D),
              _full_spec(D, D), _full_spec(1, D)],
    out_specs=_row_spec(D),
    out_shape=jax.ShapeDtypeStruct((N, D), jnp.float32),
)


def kernel(x, edge_index, W1_l, b1, W1_r, W2_l, b2, W2_r, W_head, b_head):
    src = edge_index[0]
    dst = edge_index[1]
    pad = EPAD - E
    src_p = jnp.concatenate([src, jnp.zeros((pad,), jnp.int32)])
    dst_p = jnp.concatenate([dst, jnp.full((pad,), N, jnp.int32)])
    zrows = jnp.zeros((CHUNK, D), jnp.float32)

    sums1, cnts = _seg_sum_deg(src_p, dst_p, x, zrows)
    rcp = _cntred(cnts)
    h1 = _layer1(sums1[0], sums1[1], rcp, x, W1_l, W1_r, b1.reshape(1, D))
    sums2 = _seg_sum(src_p, dst_p, h1)
    wh = jnp.zeros((D, D), jnp.float32).at[:, :3].set(W_head)
    bh = jnp.zeros((1, D), jnp.float32).at[0, :3].set(b_head)
    out = _layer2(sums2[0], sums2[1], rcp, h1, W2_l, W2_r, b2.reshape(1, D),
                  wh, bh)
    return out[:, :3]


# speedup vs baseline: 2.4211x; 1.2055x over previous
import functools

import jax
import jax.numpy as jnp
from jax import lax
from jax.experimental import pallas as pl
from jax.experimental.pallas import tpu as pltpu
from jax.experimental.pallas import tpu_sc as plsc

N = 10000
E = 320000
D = 128
NC = 2
NS = 16
NW = NC * NS
CHUNK = 128
TOTCH = E // CHUNK
NCH0 = 113
XT0 = TOTCH - NS * (NCH0 + 43)
NCH1 = 43
NPAD = 10240
STRIPE = NPAD // NS

_mesh = plsc.VectorSubcoreMesh(core_axis_name="c", subcore_axis_name="s",
                               num_cores=NC, num_subcores=NS)


def _zero_fill(buf, nrows, ncols):
    z16 = jnp.zeros((16,), jnp.float32)

    def fill(i, _):
        buf[i // (ncols // 16), pl.ds((i % (ncols // 16)) * 16, 16)] = z16
        return 0

    lax.fori_loop(0, nrows * (ncols // 16), fill, 0)


def _seg_pipe(ei_hbm, feat_hbm, out_sum, sum_sh, sidx, didx, rows,
              sems, zrows_hbm=None, cnt_pack=None):
    c = lax.axis_index("c")
    s = lax.axis_index("s")

    if zrows_hbm is None:
        _zero_fill(rows[0], CHUNK, D)
    else:
        pltpu.sync_copy(zrows_hbm, rows[0])
    if cnt_pack is not None:
        out_cnt, cntv = cnt_pack
        z16 = jnp.zeros((16,), jnp.float32)

        def czf(i, _):
            cntv[pl.ds(i * 16, 16)] = z16
            return 0

        lax.fori_loop(0, NPAD // 16, czf, 0)

    row0 = s * STRIPE

    def zcopy(j, _):
        pltpu.sync_copy(rows[0], sum_sh.at[pl.ds(row0 + j * CHUNK, CHUNK), :])
        return 0

    lax.fori_loop(0, STRIPE // CHUNK, zcopy, 0)
    plsc.subcore_barrier()

    nch = jnp.where(c == 0, NCH0 + (s < XT0), NCH1)
    cbase = jnp.where(c == 0, s * NCH0 + jnp.minimum(s, XT0),
                      NS * NCH0 + XT0 + s * NCH1)
    one16 = jnp.ones((16,), jnp.float32)

    def load_and_fire(i, par):
        base = (cbase + i) * CHUNK
        pltpu.sync_copy(ei_hbm.at[0, pl.ds(base, CHUNK)], sidx[par])
        pltpu.sync_copy(ei_hbm.at[1, pl.ds(base, CHUNK)], didx[par])
        return pltpu.async_copy(feat_hbm.at[sidx[par]], rows[par], sems[par])

    def drain_scatter(par):
        pltpu.make_async_copy(feat_hbm.at[sidx[par]], rows[par],
                              sems[par]).wait()
        pltpu.sync_copy(rows[par], sum_sh.at[didx[par]], add=True)
        if cnt_pack is not None:
            def cadd(j, _):
                v = didx[par][pl.ds(j * 16, 16)]
                plsc.addupdate_scatter(cntv, [v], one16)
                return 0

            lax.fori_loop(0, CHUNK // 16, cadd, 0)

    load_and_fire(0, 0)

    def epair(p, _):
        i0 = 2 * p

        @pl.when(i0 + 1 < nch)
        def _():
            load_and_fire(i0 + 1, 1)

        drain_scatter(0)

        @pl.when(i0 + 1 < nch)
        def _():
            @pl.when(i0 + 2 < nch)
            def _():
                load_and_fire(i0 + 2, 0)

            drain_scatter(1)

        return 0

    lax.fori_loop(0, (nch + 1) // 2, epair, 0)
    plsc.subcore_barrier()

    pltpu.sync_copy(sum_sh.at[pl.ds(row0, STRIPE), :],
                    out_sum.at[c, pl.ds(row0, STRIPE), :])
    if cnt_pack is not None:
        pltpu.sync_copy(cntv, out_cnt.at[c * NS + s])


@functools.partial(
    pl.kernel,
    out_type=(jax.ShapeDtypeStruct((NC, NPAD, D), jnp.float32),
              jax.ShapeDtypeStruct((NW, NPAD), jnp.float32)),
    mesh=_mesh,
    scratch_types=dict(
        sidx0=pltpu.VMEM((CHUNK,), jnp.int32),
        sidx1=pltpu.VMEM((CHUNK,), jnp.int32),
        didx0=pltpu.VMEM((CHUNK,), jnp.int32),
        didx1=pltpu.VMEM((CHUNK,), jnp.int32),
        rows0=pltpu.VMEM((CHUNK, D), jnp.float32),
        rows1=pltpu.VMEM((CHUNK, D), jnp.float32),
        cntv=pltpu.VMEM((NPAD,), jnp.float32),
        sum_sh=pltpu.VMEM_SHARED((NPAD, D), jnp.float32),
        gsem0=pltpu.SemaphoreType.DMA,
        gsem1=pltpu.SemaphoreType.DMA,
    ),
    compiler_params=pltpu.CompilerParams(needs_layout_passes=False),
)
def _seg_sum_deg(ei_hbm, feat_hbm, zrows_hbm, out_sum, out_cnt,
                 sidx0, sidx1, didx0, didx1, rows0, rows1, cntv, sum_sh,
                 gsem0, gsem1):
    _seg_pipe(ei_hbm, feat_hbm, out_sum, sum_sh,
              (sidx0, sidx1), (didx0, didx1), (rows0, rows1), (gsem0, gsem1),
              zrows_hbm=zrows_hbm, cnt_pack=(out_cnt, cntv))


@functools.partial(
    pl.kernel,
    out_type=jax.ShapeDtypeStruct((NC, NPAD, D), jnp.float32),
    mesh=_mesh,
    scratch_types=dict(
        sidx0=pltpu.VMEM((CHUNK,), jnp.int32),
        sidx1=pltpu.VMEM((CHUNK,), jnp.int32),
        didx0=pltpu.VMEM((CHUNK,), jnp.int32),
        didx1=pltpu.VMEM((CHUNK,), jnp.int32),
        rows0=pltpu.VMEM((CHUNK, D), jnp.float32),
        rows1=pltpu.VMEM((CHUNK, D), jnp.float32),
        sum_sh=pltpu.VMEM_SHARED((NPAD, D), jnp.float32),
        gsem0=pltpu.SemaphoreType.DMA,
        gsem1=pltpu.SemaphoreType.DMA,
    ),
)
def _seg_sum(ei_hbm, feat_hbm, out_sum,
             sidx0, sidx1, didx0, didx1, rows0, rows1, sum_sh, gsem0, gsem1):
    _seg_pipe(ei_hbm, feat_hbm, out_sum, sum_sh,
              (sidx0, sidx1), (didx0, didx1), (rows0, rows1), (gsem0, gsem1))


BN = 1000
CB = 2560


def _cntred_body(cn, out):
    out[...] = (1.0 / jnp.maximum(jnp.sum(cn[...], axis=0), 1.0))[:, None]


def _layer1_body(p0, p1, rcp, x, wl, wr, b, out):
    mean = (p0[...] + p1[...]) * rcp[...]
    acc = jnp.dot(mean, wl[...], preferred_element_type=jnp.float32)
    acc = acc + jnp.dot(x[...], wr[...], preferred_element_type=jnp.float32)
    out[...] = jnp.maximum(acc + b[...], 0.0)


def _layer2_body(q0, q1, rcp, h, wl, wr, b, wh, bh, out):
    mean = (q0[...] + q1[...]) * rcp[...]
    acc = jnp.dot(mean, wl[...], preferred_element_type=jnp.float32)
    acc = acc + jnp.dot(h[...], wr[...], preferred_element_type=jnp.float32)
    h2 = jnp.maximum(acc + b[...], 0.0)
    out[...] = jnp.dot(h2, wh[...],
                       preferred_element_type=jnp.float32) + bh[...]


DOUT = 3


def _row_spec(w):
    return pl.BlockSpec((BN, w), lambda i: (i, 0))


def _rcp_spec():
    return pl.BlockSpec((BN, 1), lambda i: (i, 0))


def _full_spec(r, cdim):
    return pl.BlockSpec((r, cdim), lambda i: (0, 0))


_cntred = pl.pallas_call(
    _cntred_body,
    grid=(NPAD // CB,),
    in_specs=[pl.BlockSpec((NW, CB), lambda i: (0, i))],
    out_specs=pl.BlockSpec((CB, 1), lambda i: (i, 0)),
    out_shape=jax.ShapeDtypeStruct((NPAD, 1), jnp.float32),
)


_layer1 = pl.pallas_call(
    _layer1_body,
    grid=(N // BN,),
    in_specs=[_row_spec(D), _row_spec(D), _rcp_spec(),
              _row_spec(D), _full_spec(D, D), _full_spec(D, D), _full_spec(1, D)],
    out_specs=_row_spec(D),
    out_shape=jax.ShapeDtypeStruct((N, D), jnp.float32),
)

_layer2 = pl.pallas_call(
    _layer2_body,
    grid=(N // BN,),
    in_specs=[_row_spec(D), _row_spec(D), _rcp_spec(),
              _row_spec(D), _full_spec(D, D), _full_spec(D, D), _full_spec(1, D),
              _full_spec(D, DOUT), _full_spec(1, DOUT)],
    out_specs=pl.BlockSpec((BN, DOUT), lambda i: (i, 0)),
    out_shape=jax.ShapeDtypeStruct((N, DOUT), jnp.float32),
)


def kernel(x, edge_index, W1_l, b1, W1_r, W2_l, b2, W2_r, W_head, b_head):
    zrows = jnp.zeros((CHUNK, D), jnp.float32)

    sums1, cnts = _seg_sum_deg(edge_index, x, zrows)
    rcp = _cntred(cnts)
    h1 = _layer1(sums1[0], sums1[1], rcp, x, W1_l, W1_r, b1.reshape(1, D))
    sums2 = _seg_sum(edge_index, h1)
    out = _layer2(sums2[0], sums2[1], rcp, h1, W2_l, W2_r, b2.reshape(1, D),
                  W_head, b_head.reshape(1, DOUT))
    return out


# speedup vs baseline: 3.1579x; 1.3043x over previous
import functools

import jax
import jax.numpy as jnp
from jax import lax
from jax.experimental import pallas as pl
from jax.experimental.pallas import tpu as pltpu
from jax.experimental.pallas import tpu_sc as plsc

N = 10000
E = 320000
D = 128
NC = 2
NS = 16
NW = NC * NS
CHUNK = 128
TOTCH = E // CHUNK
NCH0 = 78
NCH1 = 78
XT0 = TOTCH - NS * (NCH0 + NCH1)
NPAD = 10240
STRIPE = NPAD // NS

_mesh = plsc.VectorSubcoreMesh(core_axis_name="c", subcore_axis_name="s",
                               num_cores=NC, num_subcores=NS)


def _zero_fill(buf, nrows, ncols):
    z16 = jnp.zeros((16,), jnp.float32)

    def fill(i, _):
        buf[i // (ncols // 16), pl.ds((i % (ncols // 16)) * 16, 16)] = z16
        return 0

    lax.fori_loop(0, nrows * (ncols // 16), fill, 0)


def _seg_pipe(ei_hbm, feat_hbm, out_sum, sum_sh, sidx, didx, rows,
              sems, zrows_hbm=None, cnt_pack=None):
    c = lax.axis_index("c")
    s = lax.axis_index("s")

    if zrows_hbm is None:
        _zero_fill(rows[0], CHUNK, D)
    else:
        pltpu.sync_copy(zrows_hbm, rows[0])
    if cnt_pack is not None:
        out_cnt, cntv = cnt_pack
        z16 = jnp.zeros((16,), jnp.float32)

        def czf(i, _):
            cntv[pl.ds(i * 16, 16)] = z16
            return 0

        lax.fori_loop(0, NPAD // 16, czf, 0)

    row0 = s * STRIPE

    def zcopy(j, _):
        pltpu.sync_copy(rows[0], sum_sh.at[pl.ds(row0 + j * CHUNK, CHUNK), :])
        return 0

    lax.fori_loop(0, STRIPE // CHUNK, zcopy, 0)
    plsc.subcore_barrier()

    nch = jnp.where(c == 0, NCH0 + (s < XT0), NCH1)
    cbase = jnp.where(c == 0, s * NCH0 + jnp.minimum(s, XT0),
                      NS * NCH0 + XT0 + s * NCH1)
    one16 = jnp.ones((16,), jnp.float32)

    def load_and_fire(i, par):
        base = (cbase + i) * CHUNK
        pltpu.sync_copy(ei_hbm.at[0, pl.ds(base, CHUNK)], sidx[par])
        pltpu.sync_copy(ei_hbm.at[1, pl.ds(base, CHUNK)], didx[par])
        return pltpu.async_copy(feat_hbm.at[sidx[par]], rows[par], sems[par])

    def drain_scatter(par):
        pltpu.make_async_copy(feat_hbm.at[sidx[par]], rows[par],
                              sems[par]).wait()
        pltpu.sync_copy(rows[par], sum_sh.at[didx[par]], add=True)
        if cnt_pack is not None:
            def cadd(j, _):
                v = didx[par][pl.ds(j * 16, 16)]
                plsc.addupdate_scatter(cntv, [v], one16)
                return 0

            lax.fori_loop(0, CHUNK // 16, cadd, 0)

    load_and_fire(0, 0)

    def epair(p, _):
        i0 = 2 * p

        @pl.when(i0 + 1 < nch)
        def _():
            load_and_fire(i0 + 1, 1)

        drain_scatter(0)

        @pl.when(i0 + 1 < nch)
        def _():
            @pl.when(i0 + 2 < nch)
            def _():
                load_and_fire(i0 + 2, 0)

            drain_scatter(1)

        return 0

    lax.fori_loop(0, (nch + 1) // 2, epair, 0)
    plsc.subcore_barrier()

    pltpu.sync_copy(sum_sh.at[pl.ds(row0, STRIPE), :],
                    out_sum.at[c, pl.ds(row0, STRIPE), :])
    if cnt_pack is not None:
        pltpu.sync_copy(cntv, out_cnt.at[c * NS + s])


@functools.partial(
    pl.kernel,
    out_type=(jax.ShapeDtypeStruct((NC, NPAD, D), jnp.float32),
              jax.ShapeDtypeStruct((NW, NPAD), jnp.float32)),
    mesh=_mesh,
    scratch_types=dict(
        sidx0=pltpu.VMEM((CHUNK,), jnp.int32),
        sidx1=pltpu.VMEM((CHUNK,), jnp.int32),
        didx0=pltpu.VMEM((CHUNK,), jnp.int32),
        didx1=pltpu.VMEM((CHUNK,), jnp.int32),
        rows0=pltpu.VMEM((CHUNK, D), jnp.float32),
        rows1=pltpu.VMEM((CHUNK, D), jnp.float32),
        cntv=pltpu.VMEM((NPAD,), jnp.float32),
        sum_sh=pltpu.VMEM_SHARED((NPAD, D), jnp.float32),
        gsem0=pltpu.SemaphoreType.DMA,
        gsem1=pltpu.SemaphoreType.DMA,
    ),
    compiler_params=pltpu.CompilerParams(needs_layout_passes=False),
)
def _seg_sum_deg(ei_hbm, feat_hbm, zrows_hbm, out_sum, out_cnt,
                 sidx0, sidx1, didx0, didx1, rows0, rows1, cntv, sum_sh,
                 gsem0, gsem1):
    _seg_pipe(ei_hbm, feat_hbm, out_sum, sum_sh,
              (sidx0, sidx1), (didx0, didx1), (rows0, rows1), (gsem0, gsem1),
              zrows_hbm=zrows_hbm, cnt_pack=(out_cnt, cntv))


@functools.partial(
    pl.kernel,
    out_type=jax.ShapeDtypeStruct((NC, NPAD, D), jnp.float32),
    mesh=_mesh,
    scratch_types=dict(
        sidx0=pltpu.VMEM((CHUNK,), jnp.int32),
        sidx1=pltpu.VMEM((CHUNK,), jnp.int32),
        didx0=pltpu.VMEM((CHUNK,), jnp.int32),
        didx1=pltpu.VMEM((CHUNK,), jnp.int32),
        rows0=pltpu.VMEM((CHUNK, D), jnp.float32),
        rows1=pltpu.VMEM((CHUNK, D), jnp.float32),
        sum_sh=pltpu.VMEM_SHARED((NPAD, D), jnp.float32),
        gsem0=pltpu.SemaphoreType.DMA,
        gsem1=pltpu.SemaphoreType.DMA,
    ),
)
def _seg_sum(ei_hbm, feat_hbm, out_sum,
             sidx0, sidx1, didx0, didx1, rows0, rows1, sum_sh, gsem0, gsem1):
    _seg_pipe(ei_hbm, feat_hbm, out_sum, sum_sh,
              (sidx0, sidx1), (didx0, didx1), (rows0, rows1), (gsem0, gsem1))


BN = 1000
CB = 2560


def _cntred_body(cn, out):
    out[...] = (1.0 / jnp.maximum(jnp.sum(cn[...], axis=0), 1.0))[:, None]


def _layer1_body(p0, p1, rcp, x, wl, wr, b, out):
    mean = (p0[...] + p1[...]) * rcp[...]
    acc = jnp.dot(mean, wl[...], preferred_element_type=jnp.float32)
    acc = acc + jnp.dot(x[...], wr[...], preferred_element_type=jnp.float32)
    out[...] = jnp.maximum(acc + b[...], 0.0)


def _layer2_body(q0, q1, rcp, h, wl, wr, b, wh, bh, out):
    mean = (q0[...] + q1[...]) * rcp[...]
    acc = jnp.dot(mean, wl[...], preferred_element_type=jnp.float32)
    acc = acc + jnp.dot(h[...], wr[...], preferred_element_type=jnp.float32)
    h2 = jnp.maximum(acc + b[...], 0.0)
    out[...] = jnp.dot(h2, wh[...],
                       preferred_element_type=jnp.float32) + bh[...]


DOUT = 3


def _row_spec(w):
    return pl.BlockSpec((BN, w), lambda i: (i, 0))


def _rcp_spec():
    return pl.BlockSpec((BN, 1), lambda i: (i, 0))


def _full_spec(r, cdim):
    return pl.BlockSpec((r, cdim), lambda i: (0, 0))


_cntred = pl.pallas_call(
    _cntred_body,
    grid=(NPAD // CB,),
    in_specs=[pl.BlockSpec((NW, CB), lambda i: (0, i))],
    out_specs=pl.BlockSpec((CB, 1), lambda i: (i, 0)),
    out_shape=jax.ShapeDtypeStruct((NPAD, 1), jnp.float32),
)


_layer1 = pl.pallas_call(
    _layer1_body,
    grid=(N // BN,),
    in_specs=[_row_spec(D), _row_spec(D), _rcp_spec(),
              _row_spec(D), _full_spec(D, D), _full_spec(D, D), _full_spec(1, D)],
    out_specs=_row_spec(D),
    out_shape=jax.ShapeDtypeStruct((N, D), jnp.float32),
)

_layer2 = pl.pallas_call(
    _layer2_body,
    grid=(N // BN,),
    in_specs=[_row_spec(D), _row_spec(D), _rcp_spec(),
              _row_spec(D), _full_spec(D, D), _full_spec(D, D), _full_spec(1, D),
              _full_spec(D, DOUT), _full_spec(1, DOUT)],
    out_specs=pl.BlockSpec((BN, DOUT), lambda i: (i, 0)),
    out_shape=jax.ShapeDtypeStruct((N, DOUT), jnp.float32),
)


def kernel(x, edge_index, W1_l, b1, W1_r, W2_l, b2, W2_r, W_head, b_head):
    zrows = jnp.zeros((CHUNK, D), jnp.float32)

    sums1, cnts = _seg_sum_deg(edge_index, x, zrows)
    rcp = _cntred(cnts)
    h1 = _layer1(sums1[0], sums1[1], rcp, x, W1_l, W1_r, b1.reshape(1, D))
    sums2 = _seg_sum(edge_index, h1)
    out = _layer2(sums2[0], sums2[1], rcp, h1, W2_l, W2_r, b2.reshape(1, D),
                  W_head, b_head.reshape(1, DOUT))
    return out
